# baseline (speedup 1.0000x reference)
import functools

import jax
import jax.numpy as jnp
from jax.experimental import pallas as pl

N_USERS = 25000
N_ITEMS = 25000
N_AUTHORS = 5000
D = 64
B = 1024

_T = 512
_NIT_PAD = ((N_ITEMS + _T - 1) // _T) * _T


def _spmm(row, col, w, x, n_rows):
    return jax.ops.segment_sum(w[:, None] * x[col], row, num_segments=n_rows)


def _prop(row, col, w, a, b):
    feats = jnp.concatenate([a, b], axis=0)
    s = _spmm(row, col, w, feats, feats.shape[0])
    light = 0.25 * feats + 0.75 * s
    return light[: a.shape[0]], light[a.shape[0]:]


def _dense_body(ua_ref, una_ref, ai0_ref, sii_ref, sia_ref, ga_ref, gna_ref,
                qt_ref, out_ref):
    ai0 = ai0_ref[...]
    sii = sii_ref[...]
    sia = sia_ref[...]
    ga = ga_ref[...]
    gna = gna_ref[...]

    def _nrm(x):
        n = jnp.sqrt(jnp.sum(x * x, axis=0, keepdims=True))
        return x / jnp.maximum(n, 1e-12)

    atom_items = 0.5 * _nrm(sii) + 0.5 * ai0
    non_atom_items = _nrm(sia)
    auth_a = _nrm(ga)
    auth_na = gna

    dot = functools.partial(jnp.dot, preferred_element_type=jnp.float32)
    u_a, u_na = ua_ref[...], una_ref[...]
    ui = jax.nn.sigmoid(dot(u_a, atom_items) + dot(u_na, non_atom_items))
    ua = jax.nn.sigmoid(dot(u_a, auth_a) + dot(u_na, auth_na))

    itf = 0.5 * (atom_items + non_atom_items)
    atf = 0.5 * (auth_a + auth_na)
    c = dot(qt_ref[...], itf)
    wgt = jax.nn.sigmoid(jnp.sum(c * atf, axis=0, keepdims=True))

    out_ref[...] = wgt * ui + (1.0 - wgt) * ua


def _dense_stage(u_a, u_na, ai0, sii, sia, ga, gna, q):
    pad = _NIT_PAD - N_ITEMS
    args = [jnp.pad(x.T, ((0, 0), (0, pad)))
            for x in (ai0, sii, sia, ga, gna)]
    grid = (_NIT_PAD // _T,)
    item_spec = pl.BlockSpec((D, _T), lambda t: (0, t))
    u_spec = pl.BlockSpec((B, D), lambda t: (0, 0))
    return pl.pallas_call(
        _dense_body,
        grid=grid,
        in_specs=[
            u_spec, u_spec,
            item_spec, item_spec, item_spec, item_spec, item_spec,
            pl.BlockSpec((D, D), lambda t: (0, 0)),
        ],
        out_specs=pl.BlockSpec((B, _T), lambda t: (0, t)),
        out_shape=jax.ShapeDtypeStruct((B, N_ITEMS), jnp.float32),
    )(u_a, u_na, *args, q.T)


def kernel(user_emb, item_emb, author_emb, q, ui_row, ui_col, ui_w,
           ua_row, ua_col, ua_w, ai_row, ai_col, ai_w, ia_row, ia_col, ia_w,
           ii_row, ii_col, ii_w, users, author_list):
    atom_users, atom_items0 = _prop(ui_row, ui_col, ui_w, user_emb, item_emb)
    non_atom_users, non_atom_authors = _prop(ua_row, ua_col, ua_w,
                                             user_emb, author_emb)
    s_ai = _spmm(ai_row, ai_col, ai_w, atom_items0, N_AUTHORS)
    s_ii = _spmm(ii_row, ii_col, ii_w, atom_items0, N_ITEMS)
    s_ia = _spmm(ia_row, ia_col, ia_w, non_atom_authors, N_ITEMS)

    ga = s_ai[author_list]
    gna = non_atom_authors[author_list]
    return _dense_stage(atom_users[users], non_atom_users[users],
                        atom_items0, s_ii, s_ia, ga, gna, q)


# speedup vs baseline: 4.5667x; 4.5667x over previous
import functools

import jax
import jax.numpy as jnp
from jax import lax
from jax.experimental import pallas as pl
from jax.experimental.pallas import tpu as pltpu
from jax.experimental.pallas import tpu_sc as plsc

N_USERS = 25000
N_ITEMS = 25000
N_AUTHORS = 5000
D = 64
B = 1024

_T = 512
_NIT_PAD = ((N_ITEMS + _T - 1) // _T) * _T

_W = 512
_CH = 128
_K = _W // _CH
_NSUB = 16
_EALIGN = _NSUB * _W


def _rpad(n):
    return -(-n // (_NSUB * _CH)) * (_NSUB * _CH)


@functools.lru_cache(maxsize=None)
def _make_spmm(e_pad, n_pad, alpha, beta, has_base):
    per_sub = e_pad // _NSUB
    n_win = per_sub // _W
    stripe = n_pad // _NSUB
    nch = stripe // _CH
    i32, f32 = jnp.int32, jnp.float32
    mesh = plsc.VectorSubcoreMesh(core_axis_name="c", subcore_axis_name="s")

    def body(x2, row2, col2, w, *rest):
        if has_base:
            (base2, out, acc, rowb, colb, cidx, wbuf, rbuf, cbuf, bbuf,
             oidx, sem) = rest
        else:
            (out, acc, rowb, colb, cidx, wbuf, rbuf, cbuf, bbuf,
             oidx, sem) = rest
            base2 = None
        cid = lax.axis_index("c")
        sid = lax.axis_index("s")
        lo = sid * stripe
        iota = lax.iota(i32, 16)
        zeros16 = jnp.zeros((16,), f32)

        def zrow(r, c):
            cbuf[r, pl.ds(0, 16)] = zeros16
            cbuf[r, pl.ds(16, 16)] = zeros16
            return c
        lax.fori_loop(0, _CH, zrow, None)

        def zch(ch, c):
            pltpu.sync_copy(cbuf, acc.at[pl.ds(lo + ch * _CH, _CH)])
            return c
        lax.fori_loop(0, nch, zch, None)
        plsc.subcore_barrier()

        def win(widx, c):
            eb = sid * per_sub + widx * _W
            rb = sid * (per_sub // _CH) + widx * _K
            pltpu.sync_copy(row2.at[pl.ds(rb, _K)], rowb)
            pltpu.sync_copy(col2.at[pl.ds(rb, _K)], colb)
            pltpu.sync_copy(w.at[pl.ds(eb, _W)], wbuf)

            def tk(k, c2):
                for g2 in range(_CH // 16):
                    v = colb[k, pl.ds(g2 * 16, 16)]
                    cidx[k, pl.ds(g2 * 16, 16)] = v * 2 + cid
                return c2
            lax.fori_loop(0, _K, tk, None)

            cps = [pltpu.async_copy(x2.at[cidx.at[k]],
                                    rbuf.at[pl.ds(k * _CH, _CH)], sem)
                   for k in range(_K)]
            for cp in cps:
                cp.wait()

            gd = lax.GatherDimensionNumbers(offset_dims=(),
                                            collapsed_slice_dims=(0,),
                                            start_index_map=(0,))

            def mg(g, c2):
                wv = wbuf[pl.ds(g * 16, 16)]
                for e2 in range(16):
                    ws = lax.gather(wv, jnp.full((16, 1), e2, i32), gd, (1,),
                                    mode=lax.GatherScatterMode.PROMISE_IN_BOUNDS)
                    r = g * 16 + e2
                    for h in range(2):
                        rbuf[r, pl.ds(h * 16, 16)] = (
                            rbuf[r, pl.ds(h * 16, 16)] * ws)
                return c2
            lax.fori_loop(0, _W // 16, mg, None)

            for k in range(_K):
                pltpu.sync_copy(rbuf.at[pl.ds(k * _CH, _CH)],
                                acc.at[rowb.at[k]], add=True)
            return c
        lax.fori_loop(0, n_win, win, None)
        plsc.subcore_barrier()

        def ech(ch, c):
            rbase = lo + ch * _CH

            def oi(q, c2):
                oidx[pl.ds(q * 16, 16)] = (rbase + q * 16 + iota) * 2 + cid
                return c2
            lax.fori_loop(0, _CH // 16, oi, None)
            pltpu.sync_copy(acc.at[pl.ds(rbase, _CH)], cbuf)
            if has_base:
                pltpu.async_copy(base2.at[oidx], bbuf, sem).wait()

            if has_base or beta != 1.0:
                def crow(r, c2):
                    for h in range(2):
                        v = cbuf[r, pl.ds(h * 16, 16)]
                        if has_base:
                            bv = bbuf[r, pl.ds(h * 16, 16)]
                            v = alpha * bv + beta * v
                        else:
                            v = beta * v
                        cbuf[r, pl.ds(h * 16, 16)] = v
                    return c2
                lax.fori_loop(0, _CH, crow, None)
            pltpu.sync_copy(cbuf, out.at[oidx])
            return c
        lax.fori_loop(0, nch, ech, None)

    scratch = [
        pltpu.VMEM_SHARED((n_pad, 32), f32),
        pltpu.VMEM((_K, _CH), i32),
        pltpu.VMEM((_K, _CH), i32),
        pltpu.VMEM((_K, _CH), i32),
        pltpu.VMEM((_W,), f32),
        pltpu.VMEM((_W, 32), f32),
        pltpu.VMEM((_CH, 32), f32),
        pltpu.VMEM((_CH, 32), f32),
        pltpu.VMEM((_CH,), i32),
        pltpu.SemaphoreType.DMA,
    ]
    return pl.kernel(
        body,
        out_type=jax.ShapeDtypeStruct((2 * n_pad, 32), f32),
        mesh=mesh,
        scratch_types=scratch,
        compiler_params=pltpu.CompilerParams(use_tc_tiling_on_sc=False),
    )


def _sc_spmm(x, row, col, w, n_rows, base=None, alpha=0.0, beta=1.0):
    n_src = x.shape[0]
    e = row.shape[0]
    e_pad = -(-e // _EALIGN) * _EALIGN
    n_pad = _rpad(n_rows)
    if e_pad != e:
        ar = jnp.arange(e_pad - e, dtype=jnp.int32)
        row = jnp.concatenate([row, ar % n_rows])
        col = jnp.concatenate([col, ar % n_src])
        w = jnp.concatenate([w, jnp.zeros((e_pad - e,), jnp.float32)])
    x2 = x.reshape(2 * n_src, 32)
    args = [x2, row.reshape(-1, _CH), col.reshape(-1, _CH), w]
    if base is not None:
        b2 = base.reshape(2 * n_rows, 32)
        args.append(jnp.pad(b2, ((0, 2 * (n_pad - n_rows)), (0, 0))))
    fn = _make_spmm(e_pad, n_pad, float(alpha), float(beta), base is not None)
    out2 = fn(*args)
    return lax.slice(out2, (0, 0), (2 * n_rows, 32)).reshape(n_rows, 64)


def _dense_body(ua_ref, una_ref, ai0_ref, sii_ref, sia_ref, ga_ref, gna_ref,
                qt_ref, out_ref):
    ai0 = ai0_ref[...]
    sii = sii_ref[...]
    sia = sia_ref[...]
    ga = ga_ref[...]
    gna = gna_ref[...]

    def _nrm(x):
        n = jnp.sqrt(jnp.sum(x * x, axis=0, keepdims=True))
        return x / jnp.maximum(n, 1e-12)

    atom_items = 0.5 * _nrm(sii) + 0.5 * ai0
    non_atom_items = _nrm(sia)
    auth_a = _nrm(ga)
    auth_na = gna

    dot = functools.partial(jnp.dot, preferred_element_type=jnp.float32)
    u_a, u_na = ua_ref[...], una_ref[...]
    ui = jax.nn.sigmoid(dot(u_a, atom_items) + dot(u_na, non_atom_items))
    ua = jax.nn.sigmoid(dot(u_a, auth_a) + dot(u_na, auth_na))

    itf = 0.5 * (atom_items + non_atom_items)
    atf = 0.5 * (auth_a + auth_na)
    c = dot(qt_ref[...], itf)
    wgt = jax.nn.sigmoid(jnp.sum(c * atf, axis=0, keepdims=True))

    out_ref[...] = wgt * ui + (1.0 - wgt) * ua


def _dense_stage(u_a, u_na, ai0, sii, sia, ga, gna, q):
    pad = _NIT_PAD - N_ITEMS
    args = [jnp.pad(x.T, ((0, 0), (0, pad)))
            for x in (ai0, sii, sia, ga, gna)]
    grid = (_NIT_PAD // _T,)
    item_spec = pl.BlockSpec((D, _T), lambda t: (0, t))
    u_spec = pl.BlockSpec((B, D), lambda t: (0, 0))
    return pl.pallas_call(
        _dense_body,
        grid=grid,
        in_specs=[
            u_spec, u_spec,
            item_spec, item_spec, item_spec, item_spec, item_spec,
            pl.BlockSpec((D, D), lambda t: (0, 0)),
        ],
        out_specs=pl.BlockSpec((B, _T), lambda t: (0, t)),
        out_shape=jax.ShapeDtypeStruct((B, N_ITEMS), jnp.float32),
    )(u_a, u_na, *args, q.T)


def kernel(user_emb, item_emb, author_emb, q, ui_row, ui_col, ui_w,
           ua_row, ua_col, ua_w, ai_row, ai_col, ai_w, ia_row, ia_col, ia_w,
           ii_row, ii_col, ii_w, users, author_list):
    feats_ui = jnp.concatenate([user_emb, item_emb], axis=0)
    feats_ua = jnp.concatenate([user_emb, author_emb], axis=0)

    light_ui = _sc_spmm(feats_ui, ui_row, ui_col, ui_w, N_USERS + N_ITEMS,
                        base=feats_ui, alpha=0.25, beta=0.75)
    light_ua = _sc_spmm(feats_ua, ua_row, ua_col, ua_w, N_USERS + N_AUTHORS,
                        base=feats_ua, alpha=0.25, beta=0.75)

    atom_users = light_ui[:N_USERS]
    atom_items0 = light_ui[N_USERS:]
    non_atom_users = light_ua[:N_USERS]
    non_atom_authors = light_ua[N_USERS:]

    s_ai = _sc_spmm(atom_items0, ai_row, ai_col, ai_w, N_AUTHORS)
    s_ii = _sc_spmm(atom_items0, ii_row, ii_col, ii_w, N_ITEMS)
    s_ia = _sc_spmm(non_atom_authors, ia_row, ia_col, ia_w, N_ITEMS)

    ga = s_ai[author_list]
    gna = non_atom_authors[author_list]
    return _dense_stage(atom_users[users], non_atom_users[users],
                        atom_items0, s_ii, s_ia, ga, gna, q)


# speedup vs baseline: 5.2516x; 1.1500x over previous
import functools

import jax
import jax.numpy as jnp
from jax import lax
from jax.experimental import pallas as pl
from jax.experimental.pallas import tpu as pltpu
from jax.experimental.pallas import tpu_sc as plsc

N_USERS = 25000
N_ITEMS = 25000
N_AUTHORS = 5000
D = 64
B = 1024

_T = 512
_NIT_PAD = ((N_ITEMS + _T - 1) // _T) * _T

_W = 256
_CH = 128
_CE = 64
_K = _W // _CH
_NSUB = 16
_EALIGN = _NSUB * _W


def _rpad(n):
    return -(-n // (_NSUB * _CE)) * (_NSUB * _CE)


@functools.lru_cache(maxsize=None)
def _make_spmm(e_pad, n_pad, alpha, beta, has_base):
    per_sub = e_pad // _NSUB
    n_win = per_sub // _W
    stripe = n_pad // _NSUB
    nch = stripe // _CE
    i32, f32 = jnp.int32, jnp.float32
    mesh = plsc.VectorSubcoreMesh(core_axis_name="c", subcore_axis_name="s")

    def body(x2, row2, col2, w, *rest):
        if has_base:
            (base2, out, acc, rowb, colb, cidx, wbuf, rbuf, cbuf, bbuf,
             oidx, rsem, csem, wsem, gsem, ssem) = rest
        else:
            (out, acc, rowb, colb, cidx, wbuf, rbuf, cbuf, bbuf,
             oidx, rsem, csem, wsem, gsem, ssem) = rest
            base2 = None
        cid = lax.axis_index("c")
        sid = lax.axis_index("s")
        lo = sid * stripe
        iota = lax.iota(i32, 16)
        zeros16 = jnp.zeros((16,), f32)

        def zrow(r, c):
            cbuf[r, pl.ds(0, 16)] = zeros16
            cbuf[r, pl.ds(16, 16)] = zeros16
            return c
        lax.fori_loop(0, _CE, zrow, None)

        def zch(ch, c):
            pltpu.sync_copy(cbuf, acc.at[pl.ds(lo + ch * _CE, _CE)])
            return c
        lax.fori_loop(0, nch, zch, None)
        plsc.subcore_barrier()

        gd = lax.GatherDimensionNumbers(offset_dims=(),
                                        collapsed_slice_dims=(0,),
                                        start_index_map=(0,))

        def fire_lin(g):
            b = lax.rem(g, 2)
            rb = sid * (per_sub // _CH) + g * _K
            eb = sid * per_sub + g * _W
            pltpu.async_copy(row2.at[pl.ds(rb, _K)], rowb.at[b], rsem)
            pltpu.async_copy(col2.at[pl.ds(rb, _K)], colb.at[b], csem)
            pltpu.async_copy(w.at[pl.ds(eb, _W)], wbuf.at[b], wsem)

        def wait_scatters(b):
            for k in range(_K):
                pltpu.make_async_copy(rbuf.at[pl.ds(k * _CH, _CH)],
                                      acc.at[rowb.at[b].at[k]], ssem).wait()

        fire_lin(0)

        def win(widx, c):
            b = lax.rem(widx, 2)
            pltpu.make_async_copy(row2.at[pl.ds(0, _K)],
                                  rowb.at[b], rsem).wait()
            pltpu.make_async_copy(col2.at[pl.ds(0, _K)],
                                  colb.at[b], csem).wait()
            pltpu.make_async_copy(w.at[pl.ds(0, _W)],
                                  wbuf.at[b], wsem).wait()

            def tk(k, c2):
                for g2 in range(_CH // 16):
                    v = colb[b, k, pl.ds(g2 * 16, 16)]
                    cidx[b, k, pl.ds(g2 * 16, 16)] = v * 2 + cid
                return c2
            lax.fori_loop(0, _K, tk, None)

            @pl.when(widx > 0)
            def _():
                wait_scatters(b)

            @pl.when(widx < n_win - 1)
            def _():
                fire_lin(widx + 1)

            descs = [pltpu.async_copy(x2.at[cidx.at[b].at[k]],
                                      rbuf.at[pl.ds(k * _CH, _CH)], gsem)
                     for k in range(_K)]
            for d in descs:
                d.wait()

            def mg(g, c2):
                wv = wbuf[b, pl.ds(g * 16, 16)]
                for e2 in range(16):
                    ws = lax.gather(wv, jnp.full((16, 1), e2, i32), gd, (1,),
                                    mode=lax.GatherScatterMode.PROMISE_IN_BOUNDS)
                    r = g * 16 + e2
                    for h in range(2):
                        rbuf[r, pl.ds(h * 16, 16)] = (
                            rbuf[r, pl.ds(h * 16, 16)] * ws)
                return c2
            lax.fori_loop(0, _W // 16, mg, None)

            for k in range(_K):
                pltpu.async_copy(rbuf.at[pl.ds(k * _CH, _CH)],
                                 acc.at[rowb.at[b].at[k]], ssem, add=True)
            return c
        lax.fori_loop(0, n_win, win, None)
        wait_scatters((n_win - 1) % 2)
        plsc.subcore_barrier()

        def ech(ch, c):
            rbase = lo + ch * _CE

            def oi(q, c2):
                oidx[pl.ds(q * 16, 16)] = (rbase + q * 16 + iota) * 2 + cid
                return c2
            lax.fori_loop(0, _CE // 16, oi, None)
            pltpu.sync_copy(acc.at[pl.ds(rbase, _CE)], cbuf)
            if has_base:
                pltpu.async_copy(base2.at[oidx], bbuf, gsem).wait()

            if has_base or beta != 1.0:
                def crow(r, c2):
                    for h in range(2):
                        v = cbuf[r, pl.ds(h * 16, 16)]
                        if has_base:
                            bv = bbuf[r, pl.ds(h * 16, 16)]
                            v = alpha * bv + beta * v
                        else:
                            v = beta * v
                        cbuf[r, pl.ds(h * 16, 16)] = v
                    return c2
                lax.fori_loop(0, _CE, crow, None)
            pltpu.sync_copy(cbuf, out.at[oidx])
            return c
        lax.fori_loop(0, nch, ech, None)

    scratch = [
        pltpu.VMEM_SHARED((n_pad, 32), f32),
        pltpu.VMEM((2, _K, _CH), i32),
        pltpu.VMEM((2, _K, _CH), i32),
        pltpu.VMEM((2, _K, _CH), i32),
        pltpu.VMEM((2, _W), f32),
        pltpu.VMEM((_W, 32), f32),
        pltpu.VMEM((_CE, 32), f32),
        pltpu.VMEM((_CE, 32), f32),
        pltpu.VMEM((_CE,), i32),
        pltpu.SemaphoreType.DMA,
        pltpu.SemaphoreType.DMA,
        pltpu.SemaphoreType.DMA,
        pltpu.SemaphoreType.DMA,
        pltpu.SemaphoreType.DMA,
    ]
    return pl.kernel(
        body,
        out_type=jax.ShapeDtypeStruct((2 * n_pad, 32), f32),
        mesh=mesh,
        scratch_types=scratch,
        compiler_params=pltpu.CompilerParams(use_tc_tiling_on_sc=False),
    )


def _sc_spmm(x, row, col, w, n_rows, base=None, alpha=0.0, beta=1.0):
    n_src = x.shape[0]
    e = row.shape[0]
    e_pad = -(-e // _EALIGN) * _EALIGN
    n_pad = _rpad(n_rows)
    if e_pad != e:
        ar = jnp.arange(e_pad - e, dtype=jnp.int32)
        row = jnp.concatenate([row, ar % n_rows])
        col = jnp.concatenate([col, ar % n_src])
        w = jnp.concatenate([w, jnp.zeros((e_pad - e,), jnp.float32)])
    x2 = x.reshape(2 * n_src, 32)
    args = [x2, row.reshape(-1, _CH), col.reshape(-1, _CH), w]
    if base is not None:
        b2 = base.reshape(2 * n_rows, 32)
        args.append(jnp.pad(b2, ((0, 2 * (n_pad - n_rows)), (0, 0))))
    fn = _make_spmm(e_pad, n_pad, float(alpha), float(beta), base is not None)
    out2 = fn(*args)
    return lax.slice(out2, (0, 0), (2 * n_rows, 32)).reshape(n_rows, 64)


def _dense_body(ua_ref, una_ref, ai0_ref, sii_ref, sia_ref, ga_ref, gna_ref,
                qt_ref, out_ref):
    ai0 = ai0_ref[...]
    sii = sii_ref[...]
    sia = sia_ref[...]
    ga = ga_ref[...]
    gna = gna_ref[...]

    def _nrm(x):
        n = jnp.sqrt(jnp.sum(x * x, axis=0, keepdims=True))
        return x / jnp.maximum(n, 1e-12)

    atom_items = 0.5 * _nrm(sii) + 0.5 * ai0
    non_atom_items = _nrm(sia)
    auth_a = _nrm(ga)
    auth_na = gna

    dot = functools.partial(jnp.dot, preferred_element_type=jnp.float32)
    u_a, u_na = ua_ref[...], una_ref[...]
    ui = jax.nn.sigmoid(dot(u_a, atom_items) + dot(u_na, non_atom_items))
    ua = jax.nn.sigmoid(dot(u_a, auth_a) + dot(u_na, auth_na))

    itf = 0.5 * (atom_items + non_atom_items)
    atf = 0.5 * (auth_a + auth_na)
    c = dot(qt_ref[...], itf)
    wgt = jax.nn.sigmoid(jnp.sum(c * atf, axis=0, keepdims=True))

    out_ref[...] = wgt * ui + (1.0 - wgt) * ua


def _dense_stage(u_a, u_na, ai0, sii, sia, ga, gna, q):
    pad = _NIT_PAD - N_ITEMS
    args = [jnp.pad(x.T, ((0, 0), (0, pad)))
            for x in (ai0, sii, sia, ga, gna)]
    grid = (_NIT_PAD // _T,)
    item_spec = pl.BlockSpec((D, _T), lambda t: (0, t))
    u_spec = pl.BlockSpec((B, D), lambda t: (0, 0))
    return pl.pallas_call(
        _dense_body,
        grid=grid,
        in_specs=[
            u_spec, u_spec,
            item_spec, item_spec, item_spec, item_spec, item_spec,
            pl.BlockSpec((D, D), lambda t: (0, 0)),
        ],
        out_specs=pl.BlockSpec((B, _T), lambda t: (0, t)),
        out_shape=jax.ShapeDtypeStruct((B, N_ITEMS), jnp.float32),
    )(u_a, u_na, *args, q.T)


def kernel(user_emb, item_emb, author_emb, q, ui_row, ui_col, ui_w,
           ua_row, ua_col, ua_w, ai_row, ai_col, ai_w, ia_row, ia_col, ia_w,
           ii_row, ii_col, ii_w, users, author_list):
    feats_ui = jnp.concatenate([user_emb, item_emb], axis=0)
    feats_ua = jnp.concatenate([user_emb, author_emb], axis=0)

    light_ui = _sc_spmm(feats_ui, ui_row, ui_col, ui_w, N_USERS + N_ITEMS,
                        base=feats_ui, alpha=0.25, beta=0.75)
    light_ua = _sc_spmm(feats_ua, ua_row, ua_col, ua_w, N_USERS + N_AUTHORS,
                        base=feats_ua, alpha=0.25, beta=0.75)

    atom_users = light_ui[:N_USERS]
    atom_items0 = light_ui[N_USERS:]
    non_atom_users = light_ua[:N_USERS]
    non_atom_authors = light_ua[N_USERS:]

    s_ai = _sc_spmm(atom_items0, ai_row, ai_col, ai_w, N_AUTHORS)
    s_ii = _sc_spmm(atom_items0, ii_row, ii_col, ii_w, N_ITEMS)
    s_ia = _sc_spmm(non_atom_authors, ia_row, ia_col, ia_w, N_ITEMS)

    ga = s_ai[author_list]
    gna = non_atom_authors[author_list]
    return _dense_stage(atom_users[users], non_atom_users[users],
                        atom_items0, s_ii, s_ia, ga, gna, q)


# speedup vs baseline: 5.5916x; 1.0648x over previous
import functools

import jax
import jax.numpy as jnp
from jax import lax
from jax.experimental import pallas as pl
from jax.experimental.pallas import tpu as pltpu
from jax.experimental.pallas import tpu_sc as plsc

N_USERS = 25000
N_ITEMS = 25000
N_AUTHORS = 5000
D = 64
B = 1024

_T = 512
_NIT_PAD = ((N_ITEMS + _T - 1) // _T) * _T

_W = 384
_CH = 128
_CE = 64
_K = _W // _CH
_NSUB = 16
_EALIGN = _NSUB * _W


def _rpad(n):
    return -(-n // (_NSUB * _CE)) * (_NSUB * _CE)


@functools.lru_cache(maxsize=None)
def _make_spmm(e_pad, n_pad, alpha, beta, has_base):
    per_sub = e_pad // _NSUB
    n_win = per_sub // _W
    stripe = n_pad // _NSUB
    nch = stripe // _CE
    i32, f32 = jnp.int32, jnp.float32
    mesh = plsc.VectorSubcoreMesh(core_axis_name="c", subcore_axis_name="s")

    def body(x2, row2, col2, w, *rest):
        if has_base:
            (base2, out, acc, rowb, colb, cidx, wbuf, rbuf, cbuf, bbuf,
             oidx, rsem, csem, wsem, gsem, ssem) = rest
        else:
            (out, acc, rowb, colb, cidx, wbuf, rbuf, cbuf, bbuf,
             oidx, rsem, csem, wsem, gsem, ssem) = rest
            base2 = None
        cid = lax.axis_index("c")
        sid = lax.axis_index("s")
        lo = sid * stripe
        iota = lax.iota(i32, 16)
        zeros16 = jnp.zeros((16,), f32)

        def zrow(r, c):
            cbuf[r, pl.ds(0, 16)] = zeros16
            cbuf[r, pl.ds(16, 16)] = zeros16
            return c
        lax.fori_loop(0, _CE, zrow, None)

        def zch(ch, c):
            pltpu.sync_copy(cbuf, acc.at[pl.ds(lo + ch * _CE, _CE)])
            return c
        lax.fori_loop(0, nch, zch, None)
        plsc.subcore_barrier()

        gd = lax.GatherDimensionNumbers(offset_dims=(),
                                        collapsed_slice_dims=(0,),
                                        start_index_map=(0,))

        def fire_lin(g):
            b = lax.rem(g, 2)
            rb = sid * (per_sub // _CH) + g * _K
            eb = sid * per_sub + g * _W
            pltpu.async_copy(row2.at[pl.ds(rb, _K)], rowb.at[b], rsem)
            pltpu.async_copy(col2.at[pl.ds(rb, _K)], colb.at[b], csem)
            pltpu.async_copy(w.at[pl.ds(eb, _W)], wbuf.at[b], wsem)

        def wait_scatters(b):
            for k in range(_K):
                pltpu.make_async_copy(rbuf.at[pl.ds(k * _CH, _CH)],
                                      acc.at[rowb.at[b].at[k]], ssem).wait()

        fire_lin(0)

        def win(widx, c):
            b = lax.rem(widx, 2)
            pltpu.make_async_copy(row2.at[pl.ds(0, _K)],
                                  rowb.at[b], rsem).wait()
            pltpu.make_async_copy(col2.at[pl.ds(0, _K)],
                                  colb.at[b], csem).wait()
            pltpu.make_async_copy(w.at[pl.ds(0, _W)],
                                  wbuf.at[b], wsem).wait()

            def tk(k, c2):
                for g2 in range(_CH // 16):
                    v = colb[b, k, pl.ds(g2 * 16, 16)]
                    cidx[b, k, pl.ds(g2 * 16, 16)] = v * 2 + cid
                return c2
            lax.fori_loop(0, _K, tk, None)

            @pl.when(widx > 0)
            def _():
                wait_scatters(b)

            @pl.when(widx < n_win - 1)
            def _():
                fire_lin(widx + 1)

            descs = [pltpu.async_copy(x2.at[cidx.at[b].at[k]],
                                      rbuf.at[pl.ds(k * _CH, _CH)], gsem)
                     for k in range(_K)]
            for d in descs:
                d.wait()

            def mg(g, c2):
                wv = wbuf[b, pl.ds(g * 16, 16)]
                for e2 in range(16):
                    ws = lax.gather(wv, jnp.full((16, 1), e2, i32), gd, (1,),
                                    mode=lax.GatherScatterMode.PROMISE_IN_BOUNDS)
                    r = g * 16 + e2
                    for h in range(2):
                        rbuf[r, pl.ds(h * 16, 16)] = (
                            rbuf[r, pl.ds(h * 16, 16)] * ws)
                return c2
            lax.fori_loop(0, _W // 16, mg, None)

            for k in range(_K):
                pltpu.async_copy(rbuf.at[pl.ds(k * _CH, _CH)],
                                 acc.at[rowb.at[b].at[k]], ssem, add=True)
            return c
        lax.fori_loop(0, n_win, win, None)
        wait_scatters((n_win - 1) % 2)
        plsc.subcore_barrier()

        def ech(ch, c):
            rbase = lo + ch * _CE

            def oi(q, c2):
                oidx[pl.ds(q * 16, 16)] = (rbase + q * 16 + iota) * 2 + cid
                return c2
            lax.fori_loop(0, _CE // 16, oi, None)
            pltpu.sync_copy(acc.at[pl.ds(rbase, _CE)], cbuf)
            if has_base:
                pltpu.async_copy(base2.at[oidx], bbuf, gsem).wait()

            if has_base or beta != 1.0:
                def crow(r, c2):
                    for h in range(2):
                        v = cbuf[r, pl.ds(h * 16, 16)]
                        if has_base:
                            bv = bbuf[r, pl.ds(h * 16, 16)]
                            v = alpha * bv + beta * v
                        else:
                            v = beta * v
                        cbuf[r, pl.ds(h * 16, 16)] = v
                    return c2
                lax.fori_loop(0, _CE, crow, None)
            pltpu.sync_copy(cbuf, out.at[oidx])
            return c
        lax.fori_loop(0, nch, ech, None)

    scratch = [
        pltpu.VMEM_SHARED((n_pad, 32), f32),
        pltpu.VMEM((2, _K, _CH), i32),
        pltpu.VMEM((2, _K, _CH), i32),
        pltpu.VMEM((2, _K, _CH), i32),
        pltpu.VMEM((2, _W), f32),
        pltpu.VMEM((_W, 32), f32),
        pltpu.VMEM((_CE, 32), f32),
        pltpu.VMEM((_CE, 32), f32),
        pltpu.VMEM((_CE,), i32),
        pltpu.SemaphoreType.DMA,
        pltpu.SemaphoreType.DMA,
        pltpu.SemaphoreType.DMA,
        pltpu.SemaphoreType.DMA,
        pltpu.SemaphoreType.DMA,
    ]
    return pl.kernel(
        body,
        out_type=jax.ShapeDtypeStruct((2 * n_pad, 32), f32),
        mesh=mesh,
        scratch_types=scratch,
        compiler_params=pltpu.CompilerParams(use_tc_tiling_on_sc=False),
    )


def _sc_spmm(x, row, col, w, n_rows, base=None, alpha=0.0, beta=1.0):
    n_src = x.shape[0]
    e = row.shape[0]
    e_pad = -(-e // _EALIGN) * _EALIGN
    n_pad = _rpad(n_rows)
    if e_pad != e:
        ar = jnp.arange(e_pad - e, dtype=jnp.int32)
        row = jnp.concatenate([row, ar % n_rows])
        col = jnp.concatenate([col, ar % n_src])
        w = jnp.concatenate([w, jnp.zeros((e_pad - e,), jnp.float32)])
    x2 = x.reshape(2 * n_src, 32)
    args = [x2, row.reshape(-1, _CH), col.reshape(-1, _CH), w]
    if base is not None:
        b2 = base.reshape(2 * n_rows, 32)
        args.append(jnp.pad(b2, ((0, 2 * (n_pad - n_rows)), (0, 0))))
    fn = _make_spmm(e_pad, n_pad, float(alpha), float(beta), base is not None)
    out2 = fn(*args)
    return lax.slice(out2, (0, 0), (2 * n_rows, 32)).reshape(n_rows, 64)


def _dense_body(ua_ref, una_ref, ai0_ref, sii_ref, sia_ref, ga_ref, gna_ref,
                qt_ref, out_ref):
    ai0 = ai0_ref[...]
    sii = sii_ref[...]
    sia = sia_ref[...]
    ga = ga_ref[...]
    gna = gna_ref[...]

    def _nrm(x):
        n = jnp.sqrt(jnp.sum(x * x, axis=0, keepdims=True))
        return x / jnp.maximum(n, 1e-12)

    atom_items = 0.5 * _nrm(sii) + 0.5 * ai0
    non_atom_items = _nrm(sia)
    auth_a = _nrm(ga)
    auth_na = gna

    dot = functools.partial(jnp.dot, preferred_element_type=jnp.float32)
    u_a, u_na = ua_ref[...], una_ref[...]
    ui = jax.nn.sigmoid(dot(u_a, atom_items) + dot(u_na, non_atom_items))
    ua = jax.nn.sigmoid(dot(u_a, auth_a) + dot(u_na, auth_na))

    itf = 0.5 * (atom_items + non_atom_items)
    atf = 0.5 * (auth_a + auth_na)
    c = dot(qt_ref[...], itf)
    wgt = jax.nn.sigmoid(jnp.sum(c * atf, axis=0, keepdims=True))

    out_ref[...] = wgt * ui + (1.0 - wgt) * ua


def _dense_stage(u_a, u_na, ai0, sii, sia, ga, gna, q):
    pad = _NIT_PAD - N_ITEMS
    args = [jnp.pad(x.T, ((0, 0), (0, pad)))
            for x in (ai0, sii, sia, ga, gna)]
    grid = (_NIT_PAD // _T,)
    item_spec = pl.BlockSpec((D, _T), lambda t: (0, t))
    u_spec = pl.BlockSpec((B, D), lambda t: (0, 0))
    return pl.pallas_call(
        _dense_body,
        grid=grid,
        in_specs=[
            u_spec, u_spec,
            item_spec, item_spec, item_spec, item_spec, item_spec,
            pl.BlockSpec((D, D), lambda t: (0, 0)),
        ],
        out_specs=pl.BlockSpec((B, _T), lambda t: (0, t)),
        out_shape=jax.ShapeDtypeStruct((B, N_ITEMS), jnp.float32),
    )(u_a, u_na, *args, q.T)


def kernel(user_emb, item_emb, author_emb, q, ui_row, ui_col, ui_w,
           ua_row, ua_col, ua_w, ai_row, ai_col, ai_w, ia_row, ia_col, ia_w,
           ii_row, ii_col, ii_w, users, author_list):
    feats_ui = jnp.concatenate([user_emb, item_emb], axis=0)
    feats_ua = jnp.concatenate([user_emb, author_emb], axis=0)

    light_ui = _sc_spmm(feats_ui, ui_row, ui_col, ui_w, N_USERS + N_ITEMS,
                        base=feats_ui, alpha=0.25, beta=0.75)
    light_ua = _sc_spmm(feats_ua, ua_row, ua_col, ua_w, N_USERS + N_AUTHORS,
                        base=feats_ua, alpha=0.25, beta=0.75)

    atom_users = light_ui[:N_USERS]
    atom_items0 = light_ui[N_USERS:]
    non_atom_users = light_ua[:N_USERS]
    non_atom_authors = light_ua[N_USERS:]

    s_ai = _sc_spmm(atom_items0, ai_row, ai_col, ai_w, N_AUTHORS)
    s_ii = _sc_spmm(atom_items0, ii_row, ii_col, ii_w, N_ITEMS)
    s_ia = _sc_spmm(non_atom_authors, ia_row, ia_col, ia_w, N_ITEMS)

    ga = s_ai[author_list]
    gna = non_atom_authors[author_list]
    return _dense_stage(atom_users[users], non_atom_users[users],
                        atom_items0, s_ii, s_ia, ga, gna, q)
